# baseline (speedup 1.0000x reference)
import jax
import jax.numpy as jnp
from jax.experimental import pallas as pl

B, T, D, E, H, K = 1, 2048, 768, 16, 64, 2
SH = H * K
N = B * T
BT = 1024


def _moe_block(x_ref, rw_ref, bias_ref, up_ref, gate_ref, down_ref,
               sg_ref, su_ref, sd_ref, out_ref):
    x = x_ref[...]
    f32 = jnp.float32
    logits = jax.lax.dot_general(x, rw_ref[...], (((1,), (1,)), ((), ())),
                                 preferred_element_type=f32)
    logits = logits + bias_ref[...]
    col = jax.lax.broadcasted_iota(jnp.int32, (BT, E), 1)
    m1 = jnp.max(logits, axis=1, keepdims=True)
    i1 = jnp.min(jnp.where(logits == m1, col, E), axis=1, keepdims=True)
    neg = jnp.float32(-jnp.inf)
    masked = jnp.where(col == i1, neg, logits)
    m2 = jnp.max(masked, axis=1, keepdims=True)
    i2 = jnp.min(jnp.where(masked == m2, col, E), axis=1, keepdims=True)
    e2 = jnp.exp(m2 - m1)
    w1 = 1.0 / (1.0 + e2)
    w2 = e2 * w1
    acc = jnp.zeros((BT, D), dtype=f32)
    for e in range(E):
        u = jnp.dot(x, up_ref[e], preferred_element_type=f32)
        g = jnp.dot(x, gate_ref[e], preferred_element_type=f32)
        we = jnp.where(i1 == e, w1, 0.0) + jnp.where(i2 == e, w2, 0.0)
        h = (g * jax.nn.sigmoid(g)) * u * we
        acc = acc + jnp.dot(h, down_ref[e], preferred_element_type=f32)
    sg = jax.lax.dot_general(x, sg_ref[...], (((1,), (1,)), ((), ())),
                             preferred_element_type=f32)
    su = jax.lax.dot_general(x, su_ref[...], (((1,), (1,)), ((), ())),
                             preferred_element_type=f32)
    sh = (sg * jax.nn.sigmoid(sg)) * su
    acc = acc + jax.lax.dot_general(sh, sd_ref[...], (((1,), (1,)), ((), ())),
                                    preferred_element_type=f32)
    out_ref[...] = acc


@jax.jit
def _moe(flat, rw, bias2, up_w, gate_w, down_w, sg_w, su_w, sd_w):
    grid = (N // BT,)
    full2 = lambda i: (0, 0)
    full3 = lambda i: (0, 0, 0)
    return pl.pallas_call(
        _moe_block,
        grid=grid,
        in_specs=[
            pl.BlockSpec((BT, D), lambda i: (i, 0)),
            pl.BlockSpec((E, D), full2),
            pl.BlockSpec((1, E), full2),
            pl.BlockSpec((E, D, H), full3),
            pl.BlockSpec((E, D, H), full3),
            pl.BlockSpec((E, H, D), full3),
            pl.BlockSpec((SH, D), full2),
            pl.BlockSpec((SH, D), full2),
            pl.BlockSpec((D, SH), full2),
        ],
        out_specs=pl.BlockSpec((BT, D), lambda i: (i, 0)),
        out_shape=jax.ShapeDtypeStruct((N, D), jnp.float32),
    )(flat, rw, bias2, up_w, gate_w, down_w, sg_w, su_w, sd_w)


def kernel(x, router_w, router_bias, up_proj, gate_proj, down_proj,
           shared_gate_w, shared_up_w, shared_down_w):
    flat = x.reshape(N, D)
    bias2 = router_bias.reshape(1, E)
    out = _moe(flat, router_w, bias2, up_proj, gate_proj, down_proj,
               shared_gate_w, shared_up_w, shared_down_w)
    return out.reshape(B, T, D)


# speedup vs baseline: 1.6240x; 1.6240x over previous
import jax
import jax.numpy as jnp
from jax.experimental import pallas as pl
from jax.experimental.pallas import tpu as pltpu

B, T, D, E, H, K = 1, 2048, 768, 16, 64, 2
SH = H * K
N = B * T
BT = 512


def _moe_block(x_ref, rw_ref, bias_ref, up_ref, gate_ref, down_ref,
               sg_ref, su_ref, sd_ref, out_ref, up_s, gate_s, down_s):
    f32 = jnp.float32
    bf = jnp.bfloat16

    @pl.when(pl.program_id(0) == 0)
    def _build_fused():
        up_s[...] = jnp.concatenate(
            [up_ref[e] for e in range(E)], axis=1).astype(bf)
        gate_s[...] = jnp.concatenate(
            [gate_ref[e] for e in range(E)], axis=1).astype(bf)
        down_s[...] = jnp.concatenate(
            [down_ref[e] for e in range(E)], axis=0).astype(bf)

    x = x_ref[...]
    logits = jax.lax.dot_general(x, rw_ref[...], (((1,), (1,)), ((), ())),
                                 preferred_element_type=f32)
    logits = logits + bias_ref[...]
    col = jax.lax.broadcasted_iota(jnp.int32, (BT, E), 1)
    m1 = jnp.max(logits, axis=1, keepdims=True)
    i1 = jnp.min(jnp.where(logits == m1, col, E), axis=1, keepdims=True)
    neg = jnp.float32(-jnp.inf)
    masked = jnp.where(col == i1, neg, logits)
    m2 = jnp.max(masked, axis=1, keepdims=True)
    i2 = jnp.min(jnp.where(masked == m2, col, E), axis=1, keepdims=True)
    e2 = jnp.exp(m2 - m1)
    w1 = 1.0 / (1.0 + e2)
    w2 = e2 * w1
    w = jnp.where(col == i1, w1, jnp.where(col == i2, w2, 0.0))
    rep = (jax.lax.broadcasted_iota(jnp.int32, (E, E * H), 1) // H ==
           jax.lax.broadcasted_iota(jnp.int32, (E, E * H), 0)).astype(f32)
    wexp = jnp.dot(w, rep, preferred_element_type=f32)
    xb = x.astype(bf)
    u = jnp.dot(xb, up_s[...], preferred_element_type=f32)
    g = jnp.dot(xb, gate_s[...], preferred_element_type=f32)
    h = (g * jax.nn.sigmoid(g)) * u * wexp
    acc = jnp.dot(h.astype(bf), down_s[...], preferred_element_type=f32)
    sg = jax.lax.dot_general(x, sg_ref[...], (((1,), (1,)), ((), ())),
                             preferred_element_type=f32)
    su = jax.lax.dot_general(x, su_ref[...], (((1,), (1,)), ((), ())),
                             preferred_element_type=f32)
    sh = (sg * jax.nn.sigmoid(sg)) * su
    acc = acc + jax.lax.dot_general(sh, sd_ref[...], (((1,), (1,)), ((), ())),
                                    preferred_element_type=f32)
    out_ref[...] = acc


@jax.jit
def _moe(flat, rw, bias2, up_w, gate_w, down_w, sg_w, su_w, sd_w):
    grid = (N // BT,)
    full2 = lambda i: (0, 0)
    full3 = lambda i: (0, 0, 0)
    return pl.pallas_call(
        _moe_block,
        grid=grid,
        in_specs=[
            pl.BlockSpec((BT, D), lambda i: (i, 0)),
            pl.BlockSpec((E, D), full2),
            pl.BlockSpec((1, E), full2),
            pl.BlockSpec((E, D, H), full3),
            pl.BlockSpec((E, D, H), full3),
            pl.BlockSpec((E, H, D), full3),
            pl.BlockSpec((SH, D), full2),
            pl.BlockSpec((SH, D), full2),
            pl.BlockSpec((D, SH), full2),
        ],
        out_specs=pl.BlockSpec((BT, D), lambda i: (i, 0)),
        out_shape=jax.ShapeDtypeStruct((N, D), jnp.float32),
        scratch_shapes=[
            pltpu.VMEM((D, E * H), jnp.bfloat16),
            pltpu.VMEM((D, E * H), jnp.bfloat16),
            pltpu.VMEM((E * H, D), jnp.bfloat16),
        ],
    )(flat, rw, bias2, up_w, gate_w, down_w, sg_w, su_w, sd_w)


def kernel(x, router_w, router_bias, up_proj, gate_proj, down_proj,
           shared_gate_w, shared_up_w, shared_down_w):
    flat = x.reshape(N, D)
    bias2 = router_bias.reshape(1, E)
    out = _moe(flat, router_w, bias2, up_proj, gate_proj, down_proj,
               shared_gate_w, shared_up_w, shared_down_w)
    return out.reshape(B, T, D)
